# baseline (speedup 1.0000x reference)
import functools

import jax
import jax.numpy as jnp
import numpy as np
from jax import lax
from jax.experimental import pallas as pl
from jax.experimental.pallas import tpu as pltpu
from jax.experimental.pallas import tpu_sc as plsc

N, E, D, L = 10000, 320000, 128, 3

NC, NS = 2, 16
NW = NC * NS
CHUNK = 40
CPT = E // (NW * CHUNK)
NB = 6
RPT = 624
RPT_LAST = N - (NS - 1) * RPT


@functools.cache
def _make_sc_agg():
    @functools.partial(
        pl.kernel,
        out_type=jax.ShapeDtypeStruct((NC, N, D), jnp.float32),
        mesh=plsc.VectorSubcoreMesh(core_axis_name="c", subcore_axis_name="s"),
        scratch_types=[
            pltpu.VMEM((CPT, CHUNK), jnp.int32),
            pltpu.VMEM((CPT, CHUNK), jnp.int32),
        ] + [pltpu.VMEM((CHUNK, D), jnp.float32) for _ in range(NB)] + [
            pltpu.VMEM_SHARED((N, D), jnp.float32),
        ] + [pltpu.SemaphoreType.DMA for _ in range(2 * NB)],
        compiler_params=pltpu.CompilerParams(use_tc_tiling_on_sc=False),
    )
    def _sc_agg(src_hbm, dst_hbm, h_hbm, zeros_hbm, out_hbm,
                sidx_v, didx_v, *bufs):
        rows = bufs[:NB]
        agg_sh = bufs[NB]
        gsem = bufs[NB + 1:2 * NB + 1]
        ssem = bufs[2 * NB + 1:]
        c = lax.axis_index("c")
        s = lax.axis_index("s")
        wid = s * NC + c

        @pl.when(s < NS - 1)
        def _():
            pltpu.sync_copy(zeros_hbm.at[pl.ds(0, RPT)],
                            agg_sh.at[pl.ds(s * RPT, RPT)])

        @pl.when(s == NS - 1)
        def _():
            pltpu.sync_copy(zeros_hbm,
                            agg_sh.at[pl.ds((NS - 1) * RPT, RPT_LAST)])
        pltpu.sync_copy(src_hbm.at[wid], sidx_v)
        pltpu.sync_copy(dst_hbm.at[wid], didx_v)
        for k in range(NB):
            pltpu.async_copy(h_hbm.at[sidx_v.at[k]], rows[k], gsem[k])
        plsc.subcore_barrier()

        NFULL = CPT // NB
        REM = CPT - NFULL * NB

        @pl.loop(0, NFULL)
        def _(u):
            t = NB * u
            for k in range(NB):
                pltpu.make_async_copy(
                    h_hbm.at[sidx_v.at[0]], rows[k], gsem[k]).wait()
                pltpu.async_copy(
                    rows[k], agg_sh.at[didx_v.at[t + k]], ssem[k], add=True)
            for k in range(NB):
                pltpu.make_async_copy(
                    rows[k], agg_sh.at[didx_v.at[0]], ssem[k]).wait()

                @pl.when(t + k + NB < CPT)
                def _():
                    pltpu.async_copy(
                        h_hbm.at[sidx_v.at[t + k + NB]], rows[k], gsem[k])

        for k in range(REM):
            pltpu.make_async_copy(
                h_hbm.at[sidx_v.at[0]], rows[k], gsem[k]).wait()
            pltpu.sync_copy(rows[k], agg_sh.at[didx_v.at[CPT - REM + k]],
                            add=True)
        plsc.subcore_barrier()

        @pl.when(s < NS - 1)
        def _():
            pltpu.sync_copy(agg_sh.at[pl.ds(s * RPT, RPT)],
                            out_hbm.at[c, pl.ds(s * RPT, RPT)])

        @pl.when(s == NS - 1)
        def _():
            pltpu.sync_copy(agg_sh.at[pl.ds((NS - 1) * RPT, RPT_LAST)],
                            out_hbm.at[c, pl.ds((NS - 1) * RPT, RPT_LAST)])

    return _sc_agg


_INV_SQRT2 = np.float32(1.0 / np.sqrt(2.0))


def _dot_t(x, w_ref):
    return lax.dot_general(x, w_ref[...], (((1,), (1,)), ((), ())),
                           preferred_element_type=jnp.float32)


def _gin_block(eps, h, p0, p1, w1_ref, b1_ref, w2_ref, b2_ref,
               lnw_ref, lnb_ref):
    z = (1.0 + eps) * h + p0 + p1
    a = jnp.maximum(_dot_t(z, w1_ref) + b1_ref[...], 0.0)
    z2 = _dot_t(a, w2_ref) + b2_ref[...]
    mu = jnp.mean(z2, axis=-1, keepdims=True)
    var = jnp.mean((z2 - mu) ** 2, axis=-1, keepdims=True)
    zn = (z2 - mu) / jnp.sqrt(var + 1e-5) * lnw_ref[...] + lnb_ref[...]
    return zn * 0.5 * (1.0 + lax.erf(zn * _INV_SQRT2))


def _layer_body(eps_ref, h_ref, p_ref, w1_ref, b1_ref, w2_ref, b2_ref,
                lnw_ref, lnb_ref, o_ref):
    h = h_ref[...]
    g = _gin_block(eps_ref[0, 0], h, p_ref[0], p_ref[1], w1_ref, b1_ref,
                   w2_ref, b2_ref, lnw_ref, lnb_ref)
    o_ref[...] = g + h


def _tc_layer(h, p, eps_i, w1t, b1, w2t, b2, lnw_i, lnb_i, block_n):
    grid = (N // block_n,)
    return pl.pallas_call(
        _layer_body,
        grid=grid,
        in_specs=[
            pl.BlockSpec(memory_space=pltpu.SMEM),
            pl.BlockSpec((block_n, D), lambda i: (i, 0)),
            pl.BlockSpec((NC, block_n, D), lambda i: (0, i, 0)),
            pl.BlockSpec((D, D), lambda i: (0, 0)),
            pl.BlockSpec((1, D), lambda i: (0, 0)),
            pl.BlockSpec((D, D), lambda i: (0, 0)),
            pl.BlockSpec((1, D), lambda i: (0, 0)),
            pl.BlockSpec((1, D), lambda i: (0, 0)),
            pl.BlockSpec((1, D), lambda i: (0, 0)),
        ],
        out_specs=pl.BlockSpec((block_n, D), lambda i: (i, 0)),
        out_shape=jax.ShapeDtypeStruct((N, D), jnp.float32),
        compiler_params=pltpu.CompilerParams(
            dimension_semantics=("arbitrary",)),
    )(eps_i, h, p, w1t, b1, w2t, b2, lnw_i, lnb_i)


def _last_body(eps_ref, h1_ref, h_ref, p_ref, w1_ref, b1_ref, w2_ref, b2_ref,
               lnw_ref, lnb_ref, lnwf_ref, lnbf_ref, wp_ref, bp_ref, o_ref):
    h = h_ref[...]
    g = _gin_block(eps_ref[0, 0], h, p_ref[0], p_ref[1], w1_ref, b1_ref,
                   w2_ref, b2_ref, lnw_ref, lnb_ref)
    ssum = h1_ref[...] + 2.0 * h + g
    mu = jnp.mean(ssum, axis=-1, keepdims=True)
    var = jnp.mean((ssum - mu) ** 2, axis=-1, keepdims=True)
    zn = (ssum - mu) / jnp.sqrt(var + 1e-5) * lnwf_ref[...] + lnbf_ref[...]
    o_ref[...] = _dot_t(zn, wp_ref) + bp_ref[...]


def _tc_last(h1, h, p, eps_i, w1, b1, w2, b2, lnw_i, lnb_i,
             lnw_f, lnb_f, wp, bp, block_n):
    grid = (N // block_n,)
    row = pl.BlockSpec((block_n, D), lambda i: (i, 0))
    cst = pl.BlockSpec((1, D), lambda i: (0, 0))
    mat = pl.BlockSpec((D, D), lambda i: (0, 0))
    return pl.pallas_call(
        _last_body,
        grid=grid,
        in_specs=[pl.BlockSpec(memory_space=pltpu.SMEM), row, row,
                  pl.BlockSpec((NC, block_n, D), lambda i: (0, i, 0)),
                  mat, cst, mat, cst, cst, cst, cst, cst, mat, cst],
        out_specs=row,
        out_shape=jax.ShapeDtypeStruct((N, D), jnp.float32),
        compiler_params=pltpu.CompilerParams(
            dimension_semantics=("arbitrary",)),
    )(eps_i, h1, h, p, w1, b1, w2, b2, lnw_i, lnb_i, lnw_f, lnb_f, wp, bp)


def kernel(x, edge_index, W1, b1, W2, b2, eps, lnw, lnb, lnw_f, lnb_f, Wp, bp):
    ei = edge_index.astype(jnp.int32).reshape(2, NW, CPT, CHUNK)
    src3d, dst3d = ei[0], ei[1]
    zeros = jnp.zeros((RPT_LAST, D), jnp.float32)
    b1r = b1.reshape(L, 1, D)
    b2r = b2.reshape(L, 1, D)
    lnwr = lnw.reshape(L, 1, D)
    lnbr = lnb.reshape(L, 1, D)
    epsr = eps.reshape(L, 1, 1)

    block_n = 400
    h = x
    hs = []
    for i in range(L - 1):
        p = _make_sc_agg()(src3d, dst3d, h, zeros)
        h = _tc_layer(h, p, epsr[i], W1[i], b1r[i], W2[i], b2r[i],
                      lnwr[i], lnbr[i], block_n)
        hs.append(h)
    p = _make_sc_agg()(src3d, dst3d, h, zeros)
    return _tc_last(hs[0], h, p, epsr[2], W1[2], b1r[2], W2[2], b2r[2],
                    lnwr[2], lnbr[2], lnw_f.reshape(1, D),
                    lnb_f.reshape(1, D), Wp, bp.reshape(1, D), block_n)


# speedup vs baseline: 1.1084x; 1.1084x over previous
import functools

import jax
import jax.numpy as jnp
import numpy as np
from jax import lax
from jax.experimental import pallas as pl
from jax.experimental.pallas import tpu as pltpu
from jax.experimental.pallas import tpu_sc as plsc

N, E, D, L = 10000, 320000, 128, 3

NC, NS = 2, 16
NW = NC * NS
CHUNK = 40
CPT = E // (NW * CHUNK)
NB = 6
RPT = 624
RPT_LAST = N - (NS - 1) * RPT


@functools.cache
def _make_sc_agg():
    @functools.partial(
        pl.kernel,
        out_type=jax.ShapeDtypeStruct((NC, N, D), jnp.float32),
        mesh=plsc.VectorSubcoreMesh(core_axis_name="c", subcore_axis_name="s"),
        scratch_types=[
            pltpu.VMEM((CPT, CHUNK), jnp.int32),
            pltpu.VMEM((CPT, CHUNK), jnp.int32),
        ] + [pltpu.VMEM((CHUNK, D), jnp.float32) for _ in range(NB)] + [
            pltpu.VMEM_SHARED((N, D), jnp.float32),
        ] + [pltpu.SemaphoreType.DMA for _ in range(2 * NB + 2)],
        compiler_params=pltpu.CompilerParams(use_tc_tiling_on_sc=False),
    )
    def _sc_agg(src_hbm, dst_hbm, h_hbm, zeros_hbm, out_hbm,
                sidx_v, didx_v, *bufs):
        rows = bufs[:NB]
        agg_sh = bufs[NB]
        gsem = bufs[NB + 1:2 * NB + 1]
        ssem = bufs[2 * NB + 1:3 * NB + 1]
        zsem, isem = bufs[3 * NB + 1], bufs[3 * NB + 2]
        c = lax.axis_index("c")
        s = lax.axis_index("s")
        wid = s * NC + c

        @pl.when(s < NS - 1)
        def _():
            pltpu.async_copy(zeros_hbm.at[pl.ds(0, RPT)],
                             agg_sh.at[pl.ds(s * RPT, RPT)], zsem)

        @pl.when(s == NS - 1)
        def _():
            pltpu.async_copy(zeros_hbm,
                             agg_sh.at[pl.ds((NS - 1) * RPT, RPT_LAST)], zsem)
        pltpu.async_copy(src_hbm.at[wid], sidx_v, isem)
        pltpu.async_copy(dst_hbm.at[wid], didx_v, isem)
        pltpu.make_async_copy(src_hbm.at[wid], sidx_v, isem).wait()
        pltpu.make_async_copy(dst_hbm.at[wid], didx_v, isem).wait()
        for k in range(NB):
            pltpu.async_copy(h_hbm.at[sidx_v.at[k]], rows[k], gsem[k])

        @pl.when(s < NS - 1)
        def _():
            pltpu.make_async_copy(zeros_hbm.at[pl.ds(0, RPT)],
                                  agg_sh.at[pl.ds(s * RPT, RPT)], zsem).wait()

        @pl.when(s == NS - 1)
        def _():
            pltpu.make_async_copy(
                zeros_hbm, agg_sh.at[pl.ds((NS - 1) * RPT, RPT_LAST)],
                zsem).wait()
        plsc.subcore_barrier()

        NFULL = CPT // NB
        REM = CPT - NFULL * NB

        @pl.loop(0, NFULL)
        def _(u):
            t = NB * u
            for k in range(NB):
                pltpu.make_async_copy(
                    h_hbm.at[sidx_v.at[0]], rows[k], gsem[k]).wait()
                pltpu.async_copy(
                    rows[k], agg_sh.at[didx_v.at[t + k]], ssem[k], add=True)
            for k in range(NB):
                pltpu.make_async_copy(
                    rows[k], agg_sh.at[didx_v.at[0]], ssem[k]).wait()

                @pl.when(t + k + NB < CPT)
                def _():
                    pltpu.async_copy(
                        h_hbm.at[sidx_v.at[t + k + NB]], rows[k], gsem[k])

        for k in range(REM):
            pltpu.make_async_copy(
                h_hbm.at[sidx_v.at[0]], rows[k], gsem[k]).wait()
            pltpu.sync_copy(rows[k], agg_sh.at[didx_v.at[CPT - REM + k]],
                            add=True)
        plsc.subcore_barrier()

        @pl.when(s < NS - 1)
        def _():
            pltpu.sync_copy(agg_sh.at[pl.ds(s * RPT, RPT)],
                            out_hbm.at[c, pl.ds(s * RPT, RPT)])

        @pl.when(s == NS - 1)
        def _():
            pltpu.sync_copy(agg_sh.at[pl.ds((NS - 1) * RPT, RPT_LAST)],
                            out_hbm.at[c, pl.ds((NS - 1) * RPT, RPT_LAST)])

    return _sc_agg


_INV_SQRT2 = np.float32(1.0 / np.sqrt(2.0))


def _dot_t(x, w_ref):
    return lax.dot_general(x, w_ref[...], (((1,), (1,)), ((), ())),
                           preferred_element_type=jnp.float32)


def _gin_block(eps, h, p0, p1, w1_ref, b1_ref, w2_ref, b2_ref,
               lnw_ref, lnb_ref):
    z = (1.0 + eps) * h + p0 + p1
    a = jnp.maximum(_dot_t(z, w1_ref) + b1_ref[...], 0.0)
    z2 = _dot_t(a, w2_ref) + b2_ref[...]
    mu = jnp.mean(z2, axis=-1, keepdims=True)
    var = jnp.mean((z2 - mu) ** 2, axis=-1, keepdims=True)
    zn = (z2 - mu) / jnp.sqrt(var + 1e-5) * lnw_ref[...] + lnb_ref[...]
    return zn * 0.5 * (1.0 + lax.erf(zn * _INV_SQRT2))


def _layer_body(eps_ref, h_ref, p_ref, w1_ref, b1_ref, w2_ref, b2_ref,
                lnw_ref, lnb_ref, o_ref):
    h = h_ref[...]
    g = _gin_block(eps_ref[0, 0], h, p_ref[0], p_ref[1], w1_ref, b1_ref,
                   w2_ref, b2_ref, lnw_ref, lnb_ref)
    o_ref[...] = g + h


def _tc_layer(h, p, eps_i, w1t, b1, w2t, b2, lnw_i, lnb_i, block_n):
    grid = (N // block_n,)
    return pl.pallas_call(
        _layer_body,
        grid=grid,
        in_specs=[
            pl.BlockSpec(memory_space=pltpu.SMEM),
            pl.BlockSpec((block_n, D), lambda i: (i, 0)),
            pl.BlockSpec((NC, block_n, D), lambda i: (0, i, 0)),
            pl.BlockSpec((D, D), lambda i: (0, 0)),
            pl.BlockSpec((1, D), lambda i: (0, 0)),
            pl.BlockSpec((D, D), lambda i: (0, 0)),
            pl.BlockSpec((1, D), lambda i: (0, 0)),
            pl.BlockSpec((1, D), lambda i: (0, 0)),
            pl.BlockSpec((1, D), lambda i: (0, 0)),
        ],
        out_specs=pl.BlockSpec((block_n, D), lambda i: (i, 0)),
        out_shape=jax.ShapeDtypeStruct((N, D), jnp.float32),
        compiler_params=pltpu.CompilerParams(
            dimension_semantics=("arbitrary",)),
    )(eps_i, h, p, w1t, b1, w2t, b2, lnw_i, lnb_i)


def _last_body(eps_ref, h1_ref, h_ref, p_ref, w1_ref, b1_ref, w2_ref, b2_ref,
               lnw_ref, lnb_ref, lnwf_ref, lnbf_ref, wp_ref, bp_ref, o_ref):
    h = h_ref[...]
    g = _gin_block(eps_ref[0, 0], h, p_ref[0], p_ref[1], w1_ref, b1_ref,
                   w2_ref, b2_ref, lnw_ref, lnb_ref)
    ssum = h1_ref[...] + 2.0 * h + g
    mu = jnp.mean(ssum, axis=-1, keepdims=True)
    var = jnp.mean((ssum - mu) ** 2, axis=-1, keepdims=True)
    zn = (ssum - mu) / jnp.sqrt(var + 1e-5) * lnwf_ref[...] + lnbf_ref[...]
    o_ref[...] = _dot_t(zn, wp_ref) + bp_ref[...]


def _tc_last(h1, h, p, eps_i, w1, b1, w2, b2, lnw_i, lnb_i,
             lnw_f, lnb_f, wp, bp, block_n):
    grid = (N // block_n,)
    row = pl.BlockSpec((block_n, D), lambda i: (i, 0))
    cst = pl.BlockSpec((1, D), lambda i: (0, 0))
    mat = pl.BlockSpec((D, D), lambda i: (0, 0))
    return pl.pallas_call(
        _last_body,
        grid=grid,
        in_specs=[pl.BlockSpec(memory_space=pltpu.SMEM), row, row,
                  pl.BlockSpec((NC, block_n, D), lambda i: (0, i, 0)),
                  mat, cst, mat, cst, cst, cst, cst, cst, mat, cst],
        out_specs=row,
        out_shape=jax.ShapeDtypeStruct((N, D), jnp.float32),
        compiler_params=pltpu.CompilerParams(
            dimension_semantics=("arbitrary",)),
    )(eps_i, h1, h, p, w1, b1, w2, b2, lnw_i, lnb_i, lnw_f, lnb_f, wp, bp)


def kernel(x, edge_index, W1, b1, W2, b2, eps, lnw, lnb, lnw_f, lnb_f, Wp, bp):
    ei = edge_index.astype(jnp.int32).reshape(2, NW, CPT, CHUNK)
    src3d, dst3d = ei[0], ei[1]
    zeros = jnp.zeros((RPT_LAST, D), jnp.float32)
    b1r = b1.reshape(L, 1, D)
    b2r = b2.reshape(L, 1, D)
    lnwr = lnw.reshape(L, 1, D)
    lnbr = lnb.reshape(L, 1, D)
    epsr = eps.reshape(L, 1, 1)

    block_n = 2000
    h = x
    hs = []
    for i in range(L - 1):
        p = _make_sc_agg()(src3d, dst3d, h, zeros)
        h = _tc_layer(h, p, epsr[i], W1[i], b1r[i], W2[i], b2r[i],
                      lnwr[i], lnbr[i], block_n)
        hs.append(h)
    p = _make_sc_agg()(src3d, dst3d, h, zeros)
    return _tc_last(hs[0], h, p, epsr[2], W1[2], b1r[2], W2[2], b2r[2],
                    lnwr[2], lnbr[2], lnw_f.reshape(1, D),
                    lnb_f.reshape(1, D), Wp, bp.reshape(1, D), block_n)
